# baseline (speedup 1.0000x reference)
import functools

import jax
import jax.numpy as jnp
from jax import lax
from jax.experimental import pallas as pl
from jax.experimental.pallas import tpu as pltpu
from jax.experimental.pallas import tpu_sc as plsc

C = 64
SIDE = 1024
HW = SIDE * SIDE
NC, NS, L = 2, 16, 16
NW = NC * NS
B = 128


_TBLK = 2048


def _transpose_body(g_ref, t_ref):
    t_ref[...] = g_ref[...].T


_transpose = pl.pallas_call(
    _transpose_body,
    grid=(HW // _TBLK,),
    in_specs=[pl.BlockSpec((C, _TBLK), lambda i: (0, i))],
    out_specs=pl.BlockSpec((_TBLK, C), lambda i: (i, 0)),
    out_shape=jax.ShapeDtypeStruct((HW, C), jnp.float32),
)


def _sc_body(nchunk, xs_hbm, ys_hbm, table_hbm, out_hbm,
             x_v, y_v, i00, i10, i01, i11, w00, w10, w01, w11,
             t00, t10, t01, t11, out_v, s0, s1, s2, s3):
    cid = lax.axis_index("c")
    sid = lax.axis_index("s")
    wid = sid * NC + cid
    wbase = wid * (nchunk * B)

    def chunk_body(g, _):
        base = wbase + g * B
        pltpu.sync_copy(xs_hbm.at[pl.ds(base, B)], x_v)
        pltpu.sync_copy(ys_hbm.at[pl.ds(base, B)], y_v)

        for j in range(B // L):
            sl = pl.ds(j * L, L)
            fx = ((x_v[sl] + 1.0) * jnp.float32(SIDE) - 1.0) * 0.5
            fy = ((y_v[sl] + 1.0) * jnp.float32(SIDE) - 1.0) * 0.5
            x0 = fx.astype(jnp.int32)
            y0 = fy.astype(jnp.int32)
            wx1 = fx - x0.astype(jnp.float32)
            wy1 = fy - y0.astype(jnp.float32)
            wx0 = 1.0 - wx1
            wy0 = 1.0 - wy1
            x1 = x0 + 1
            y1 = y0 + 1
            wx1 = jnp.where(x1 <= SIDE - 1, wx1, 0.0)
            wy1 = jnp.where(y1 <= SIDE - 1, wy1, 0.0)
            x1c = jnp.minimum(x1, SIDE - 1)
            y1c = jnp.minimum(y1, SIDE - 1)
            r0 = y0 * SIDE
            r1 = y1c * SIDE
            i00[sl] = r0 + x0
            i10[sl] = r0 + x1c
            i01[sl] = r1 + x0
            i11[sl] = r1 + x1c
            w00[sl] = wx0 * wy0
            w10[sl] = wx1 * wy0
            w01[sl] = wx0 * wy1
            w11[sl] = wx1 * wy1

        cp0 = pltpu.async_copy(table_hbm.at[i00], t00, s0)
        cp1 = pltpu.async_copy(table_hbm.at[i10], t10, s1)
        cp2 = pltpu.async_copy(table_hbm.at[i01], t01, s2)
        cp3 = pltpu.async_copy(table_hbm.at[i11], t11, s3)
        cp0.wait()
        cp1.wait()
        cp2.wait()
        cp3.wait()

        def blend_body(jg, _):
            sl = pl.ds(jg * L, L)
            wv00 = w00[sl]
            wv10 = w10[sl]
            wv01 = w01[sl]
            wv11 = w11[sl]
            for lane in range(L):
                i = jg * L + lane
                a00 = jnp.full((L,), wv00[lane], jnp.float32)
                a10 = jnp.full((L,), wv10[lane], jnp.float32)
                a01 = jnp.full((L,), wv01[lane], jnp.float32)
                a11 = jnp.full((L,), wv11[lane], jnp.float32)
                for k in range(C // L):
                    sk = pl.ds(k * L, L)
                    acc = (t00[i, sk] * a00 + t10[i, sk] * a10
                           + t01[i, sk] * a01 + t11[i, sk] * a11)
                    out_v[i, sk] = acc
            return ()

        lax.fori_loop(0, B // L, blend_body, ())
        pltpu.sync_copy(out_v, out_hbm.at[pl.ds(base, B)])
        return ()

    lax.fori_loop(0, nchunk, chunk_body, ())


def _make_sc_sample(npad):
    nchunk = npad // (NW * B)
    mesh = plsc.VectorSubcoreMesh(
        core_axis_name="c", subcore_axis_name="s",
        num_cores=NC, num_subcores=NS)
    return pl.kernel(
        functools.partial(_sc_body, nchunk),
        out_type=jax.ShapeDtypeStruct((npad, C), jnp.float32),
        mesh=mesh,
        compiler_params=pltpu.CompilerParams(use_tc_tiling_on_sc=False),
        scratch_types=[
            pltpu.VMEM((B,), jnp.float32),
            pltpu.VMEM((B,), jnp.float32),
            pltpu.VMEM((B,), jnp.int32),
            pltpu.VMEM((B,), jnp.int32),
            pltpu.VMEM((B,), jnp.int32),
            pltpu.VMEM((B,), jnp.int32),
            pltpu.VMEM((B,), jnp.float32),
            pltpu.VMEM((B,), jnp.float32),
            pltpu.VMEM((B,), jnp.float32),
            pltpu.VMEM((B,), jnp.float32),
            pltpu.VMEM((B, C), jnp.float32),
            pltpu.VMEM((B, C), jnp.float32),
            pltpu.VMEM((B, C), jnp.float32),
            pltpu.VMEM((B, C), jnp.float32),
            pltpu.VMEM((B, C), jnp.float32),
            pltpu.SemaphoreType.DMA,
            pltpu.SemaphoreType.DMA,
            pltpu.SemaphoreType.DMA,
            pltpu.SemaphoreType.DMA,
        ],
    )


def kernel(coords, grid):
    n = coords.shape[0]
    step = NW * B
    npad = ((n + step - 1) // step) * step
    table = _transpose(grid.reshape(C, HW))
    xs = coords[:, 0]
    ys = coords[:, 1]
    if npad != n:
        pad = jnp.full((npad - n,), 0.25, jnp.float32)
        xs = jnp.concatenate([xs, pad])
        ys = jnp.concatenate([ys, pad])
    out = _make_sc_sample(npad)(xs, ys, table)
    return out[:n], coords


# speedup vs baseline: 1.2900x; 1.2900x over previous
import functools

import jax
import jax.numpy as jnp
from jax import lax
from jax.experimental import pallas as pl
from jax.experimental.pallas import tpu as pltpu
from jax.experimental.pallas import tpu_sc as plsc

C = 64
SIDE = 1024
HW = SIDE * SIDE
NC, NS, L = 2, 16, 16
NW = NC * NS
B = 128


def _sc_body(nchunk, xs_hbm, ys_hbm, table_hbm, out_hbm,
             x_v, y_v, i00, i10, i01, i11, w00, w10, w01, w11,
             t00, t10, t01, t11, out_v, s0, s1, s2, s3):
    cid = lax.axis_index("c")
    sid = lax.axis_index("s")
    wid = sid * NC + cid
    wbase = wid * (nchunk * B)

    def chunk_body(g, _):
        base = wbase + g * B
        pltpu.sync_copy(xs_hbm.at[pl.ds(base, B)], x_v)
        pltpu.sync_copy(ys_hbm.at[pl.ds(base, B)], y_v)

        for j in range(B // L):
            sl = pl.ds(j * L, L)
            fx = ((x_v[sl] + 1.0) * jnp.float32(SIDE) - 1.0) * 0.5
            fy = ((y_v[sl] + 1.0) * jnp.float32(SIDE) - 1.0) * 0.5
            x0 = fx.astype(jnp.int32)
            y0 = fy.astype(jnp.int32)
            wx1 = fx - x0.astype(jnp.float32)
            wy1 = fy - y0.astype(jnp.float32)
            wx0 = 1.0 - wx1
            wy0 = 1.0 - wy1
            x1 = x0 + 1
            y1 = y0 + 1
            wx1 = jnp.where(x1 <= SIDE - 1, wx1, 0.0)
            wy1 = jnp.where(y1 <= SIDE - 1, wy1, 0.0)
            x1c = jnp.minimum(x1, SIDE - 1)
            y1c = jnp.minimum(y1, SIDE - 1)
            r0 = y0 * SIDE
            r1 = y1c * SIDE
            i00[sl] = r0 + x0
            i10[sl] = r0 + x1c
            i01[sl] = r1 + x0
            i11[sl] = r1 + x1c
            w00[sl] = wx0 * wy0
            w10[sl] = wx1 * wy0
            w01[sl] = wx0 * wy1
            w11[sl] = wx1 * wy1

        cp0 = pltpu.async_copy(table_hbm.at[i00], t00, s0)
        cp1 = pltpu.async_copy(table_hbm.at[i10], t10, s1)
        cp2 = pltpu.async_copy(table_hbm.at[i01], t01, s2)
        cp3 = pltpu.async_copy(table_hbm.at[i11], t11, s3)
        cp0.wait()
        cp1.wait()
        cp2.wait()
        cp3.wait()

        def blend_body(jg, _):
            sl = pl.ds(jg * L, L)
            wv00 = w00[sl]
            wv10 = w10[sl]
            wv01 = w01[sl]
            wv11 = w11[sl]
            for lane in range(L):
                i = jg * L + lane
                a00 = jnp.full((L,), wv00[lane], jnp.float32)
                a10 = jnp.full((L,), wv10[lane], jnp.float32)
                a01 = jnp.full((L,), wv01[lane], jnp.float32)
                a11 = jnp.full((L,), wv11[lane], jnp.float32)
                for k in range(C // L):
                    sk = pl.ds((lane % 2) * C + k * L, L)
                    acc = (t00[i, sk2 := pl.ds(k * L, L)] * a00
                           + t10[i, sk2] * a10
                           + t01[i, sk2] * a01 + t11[i, sk2] * a11)
                    out_v[jg * (L // 2) + lane // 2, sk] = acc
            return ()

        lax.fori_loop(0, B // L, blend_body, ())
        pltpu.sync_copy(out_v, out_hbm.at[pl.ds(base * (C // 128), B * C // 128)])
        return ()

    lax.fori_loop(0, nchunk, chunk_body, ())


def _make_sc_sample(npad):
    nchunk = npad // (NW * B)
    mesh = plsc.VectorSubcoreMesh(
        core_axis_name="c", subcore_axis_name="s",
        num_cores=NC, num_subcores=NS)
    return pl.kernel(
        functools.partial(_sc_body, nchunk),
        out_type=jax.ShapeDtypeStruct((npad * C // 128, 128), jnp.float32),
        mesh=mesh,
        compiler_params=pltpu.CompilerParams(use_tc_tiling_on_sc=False),
        scratch_types=[
            pltpu.VMEM((B,), jnp.float32),
            pltpu.VMEM((B,), jnp.float32),
            pltpu.VMEM((B,), jnp.int32),
            pltpu.VMEM((B,), jnp.int32),
            pltpu.VMEM((B,), jnp.int32),
            pltpu.VMEM((B,), jnp.int32),
            pltpu.VMEM((B,), jnp.float32),
            pltpu.VMEM((B,), jnp.float32),
            pltpu.VMEM((B,), jnp.float32),
            pltpu.VMEM((B,), jnp.float32),
            pltpu.VMEM((B, C), jnp.float32),
            pltpu.VMEM((B, C), jnp.float32),
            pltpu.VMEM((B, C), jnp.float32),
            pltpu.VMEM((B, C), jnp.float32),
            pltpu.VMEM((B * C // 128, 128), jnp.float32),
            pltpu.SemaphoreType.DMA,
            pltpu.SemaphoreType.DMA,
            pltpu.SemaphoreType.DMA,
            pltpu.SemaphoreType.DMA,
        ],
    )


def kernel(coords, grid):
    n = coords.shape[0]
    step = NW * B
    npad = ((n + step - 1) // step) * step
    table = grid.reshape(C, HW).T
    xs = coords[:, 0]
    ys = coords[:, 1]
    if npad != n:
        pad = jnp.full((npad - n,), 0.25, jnp.float32)
        xs = jnp.concatenate([xs, pad])
        ys = jnp.concatenate([ys, pad])
    out = _make_sc_sample(npad)(xs, ys, table)
    return out.reshape(npad, C)[:n], coords
